# initial kernel scaffold (unmeasured)
import jax
import jax.numpy as jnp
from jax import lax
from jax.experimental import pallas as pl
from jax.experimental.pallas import tpu as pltpu

N_DEV = 4
N_HOPS = N_DEV - 1
N_LAYERS = 3


def kernel(x, Win0, Wout0, Win1, Wout1, Win2, Wout2):
    b, d_shard = x.shape
    h_dim = Win0.shape[1]
    out_cols = Wout0.shape[1]

    def body(x_ref, win0_ref, wout0_ref, win1_ref, wout1_ref,
             win2_ref, wout2_ref, out_ref,
             comm, send_sems, recv_sems, layer_sems):
        my = lax.axis_index("i")
        left = lax.rem(my + N_DEV - 1, N_DEV)
        right = lax.rem(my + 1, N_DEV)

        barrier = pltpu.get_barrier_semaphore()
        for nbr in (left, right):
            pl.semaphore_signal(barrier, inc=1, device_id=(nbr,),
                                device_id_type=pl.DeviceIdType.MESH)
        pl.semaphore_wait(barrier, 2)

        wins = [win0_ref, win1_ref, win2_ref]
        wouts = [wout0_ref, wout1_ref, wout2_ref]

        x_val = x_ref[...]
        for l in range(N_LAYERS):
            partial = jnp.dot(x_val, wins[l][...],
                              preferred_element_type=jnp.float32)
            comm[0] = partial
            acc = partial
            for h in range(N_HOPS):
                rdma = pltpu.make_async_remote_copy(
                    src_ref=comm.at[h],
                    dst_ref=comm.at[h + 1],
                    send_sem=send_sems.at[h],
                    recv_sem=recv_sems.at[h],
                    device_id=(right,),
                    device_id_type=pl.DeviceIdType.MESH,
                )
                rdma.start()
                rdma.wait()
                acc = acc + comm[h + 1]

            h_relu = jnp.maximum(acc, 0.0)
            x_val = jnp.dot(h_relu, wouts[l][...],
                            preferred_element_type=jnp.float32)

            for nbr in (left, right):
                pl.semaphore_signal(layer_sems.at[l], inc=1, device_id=(nbr,),
                                    device_id_type=pl.DeviceIdType.MESH)
            pl.semaphore_wait(layer_sems.at[l], 2)

        out_ref[...] = x_val

    return pl.pallas_call(
        body,
        out_shape=jax.ShapeDtypeStruct((b, out_cols), jnp.float32),
        in_specs=[pl.BlockSpec(memory_space=pltpu.VMEM)] * 7,
        out_specs=pl.BlockSpec(memory_space=pltpu.VMEM),
        scratch_shapes=[
            pltpu.VMEM((N_HOPS + 1, b, h_dim), jnp.float32),
            pltpu.SemaphoreType.DMA((N_HOPS,)),
            pltpu.SemaphoreType.DMA((N_HOPS,)),
            pltpu.SemaphoreType.REGULAR((N_LAYERS,)),
        ],
        compiler_params=pltpu.CompilerParams(collective_id=0),
    )(x, Win0, Wout0, Win1, Wout1, Win2, Wout2)


# baseline (device time: 94523 ns/iter reference)
import jax
import jax.numpy as jnp
from jax import lax
from jax.experimental import pallas as pl
from jax.experimental.pallas import tpu as pltpu

N_DEV = 4
N_HOPS = N_DEV - 1
N_LAYERS = 3


def kernel(x, Win0, Wout0, Win1, Wout1, Win2, Wout2):
    b, d_shard = x.shape
    h_dim = Win0.shape[1]
    out_cols = Wout0.shape[1]

    def body(x_ref, win0_ref, wout0_ref, win1_ref, wout1_ref,
             win2_ref, wout2_ref, out_ref,
             comm, send_sems, recv_sems, layer_sems):
        my = lax.axis_index("i")
        left = lax.rem(my + N_DEV - 1, N_DEV)
        right = lax.rem(my + 1, N_DEV)

        barrier = pltpu.get_barrier_semaphore()
        for nbr in (left, right):
            pl.semaphore_signal(barrier, inc=1, device_id=(nbr,),
                                device_id_type=pl.DeviceIdType.MESH)
        pl.semaphore_wait(barrier, 2)

        wins = [win0_ref, win1_ref, win2_ref]
        wouts = [wout0_ref, wout1_ref, wout2_ref]

        x_val = x_ref[...]
        for l in range(N_LAYERS):
            partial = jnp.dot(x_val, wins[l][...],
                              preferred_element_type=jnp.float32)
            comm[0] = partial
            acc = partial
            for h in range(N_HOPS):
                rdma = pltpu.make_async_remote_copy(
                    src_ref=comm.at[h],
                    dst_ref=comm.at[h + 1],
                    send_sem=send_sems.at[h],
                    recv_sem=recv_sems.at[h],
                    device_id=(right,),
                    device_id_type=pl.DeviceIdType.MESH,
                )
                rdma.start()
                rdma.wait()
                acc = acc + comm[h + 1]

            h_relu = jnp.maximum(acc, 0.0)
            x_val = jnp.dot(h_relu, wouts[l][...],
                            preferred_element_type=jnp.float32)

            for nbr in (left, right):
                pl.semaphore_signal(layer_sems.at[l], inc=1, device_id=(nbr,),
                                    device_id_type=pl.DeviceIdType.MESH)
            pl.semaphore_wait(layer_sems.at[l], 2)

        out_ref[...] = x_val

    return pl.pallas_call(
        body,
        out_shape=jax.ShapeDtypeStruct((b, out_cols), jnp.float32),
        in_specs=[pl.BlockSpec(memory_space=pltpu.VMEM)] * 7,
        out_specs=pl.BlockSpec(memory_space=pltpu.VMEM),
        scratch_shapes=[
            pltpu.VMEM((N_HOPS + 1, b, h_dim), jnp.float32),
            pltpu.SemaphoreType.DMA((N_HOPS,)),
            pltpu.SemaphoreType.DMA((N_HOPS,)),
            pltpu.SemaphoreType.REGULAR((N_LAYERS,)),
        ],
        compiler_params=pltpu.CompilerParams(
            collective_id=0,
            vmem_limit_bytes=100 * 1024 * 1024,
        ),
    )(x, Win0, Wout0, Win1, Wout1, Win2, Wout2)


# device time: 55620 ns/iter; 1.6994x vs baseline; 1.6994x over previous
import jax
import jax.numpy as jnp
from jax import lax
from jax.experimental import pallas as pl
from jax.experimental.pallas import tpu as pltpu

N_DEV = 4
N_PEERS = N_DEV - 1
N_LAYERS = 3


def kernel(x, Win0, Wout0, Win1, Wout1, Win2, Wout2):
    b, d_shard = x.shape
    h_dim = Win0.shape[1]
    out_cols = Wout0.shape[1]
    q = h_dim // N_DEV

    def body(x_ref, win0_ref, wout0_ref, win1_ref, wout1_ref,
             win2_ref, wout2_ref, out_ref,
             partial, rs_buf, myq, ag_buf,
             rs_send_sems, rs_recv_sems, ag_send_sems, ag_recv_sems):
        my = lax.axis_index("i")
        peers = [lax.rem(my + o, N_DEV) for o in range(1, N_DEV)]

        barrier = pltpu.get_barrier_semaphore()
        for p in peers:
            pl.semaphore_signal(barrier, inc=1, device_id=(p,),
                                device_id_type=pl.DeviceIdType.MESH)
        pl.semaphore_wait(barrier, N_PEERS)

        wins = [win0_ref, win1_ref, win2_ref]
        wouts = [wout0_ref, wout1_ref, wout2_ref]

        def rs_recv_desc(k):
            return pltpu.make_async_remote_copy(
                src_ref=rs_buf.at[k], dst_ref=rs_buf.at[k],
                send_sem=rs_send_sems.at[k], recv_sem=rs_recv_sems.at[k],
                device_id=(my,), device_id_type=pl.DeviceIdType.MESH)

        def ag_recv_desc(k):
            return pltpu.make_async_remote_copy(
                src_ref=ag_buf.at[k], dst_ref=ag_buf.at[k],
                send_sem=ag_send_sems.at[k], recv_sem=ag_recv_sems.at[k],
                device_id=(my,), device_id_type=pl.DeviceIdType.MESH)

        x_val = x_ref[...]
        for l in range(N_LAYERS):
            partial[...] = jnp.dot(x_val, wins[l][...],
                                   preferred_element_type=jnp.float32)

            rs_sends = []
            for o, p in zip(range(1, N_DEV), peers):
                rdma = pltpu.make_async_remote_copy(
                    src_ref=partial.at[:, pl.ds(p * q, q)],
                    dst_ref=rs_buf.at[N_PEERS - o],
                    send_sem=rs_send_sems.at[o - 1],
                    recv_sem=rs_recv_sems.at[N_PEERS - o],
                    device_id=(p,), device_id_type=pl.DeviceIdType.MESH)
                rdma.start()
                rs_sends.append(rdma)

            acc = partial[:, pl.ds(my * q, q)]
            for k in range(N_PEERS):
                rs_recv_desc(k).wait_recv()
                acc = acc + rs_buf[k]
            myq[...] = jnp.maximum(acc, 0.0)

            ag_sends = []
            for o, p in zip(range(1, N_DEV), peers):
                rdma = pltpu.make_async_remote_copy(
                    src_ref=myq,
                    dst_ref=ag_buf.at[N_PEERS - o],
                    send_sem=ag_send_sems.at[o - 1],
                    recv_sem=ag_recv_sems.at[N_PEERS - o],
                    device_id=(p,), device_id_type=pl.DeviceIdType.MESH)
                rdma.start()
                ag_sends.append(rdma)

            wout = wouts[l]
            x_val = jnp.dot(myq[...], wout[pl.ds(my * q, q), :],
                            preferred_element_type=jnp.float32)
            for k in range(N_PEERS):
                ag_recv_desc(k).wait_recv()
                s = lax.rem(my + k + 1, N_DEV)
                x_val = x_val + jnp.dot(ag_buf[k], wout[pl.ds(s * q, q), :],
                                        preferred_element_type=jnp.float32)

            for rdma in rs_sends + ag_sends:
                rdma.wait_send()

        out_ref[...] = x_val

    return pl.pallas_call(
        body,
        out_shape=jax.ShapeDtypeStruct((b, out_cols), jnp.float32),
        in_specs=[pl.BlockSpec(memory_space=pltpu.VMEM)] * 7,
        out_specs=pl.BlockSpec(memory_space=pltpu.VMEM),
        scratch_shapes=[
            pltpu.VMEM((b, h_dim), jnp.float32),
            pltpu.VMEM((N_PEERS, b, q), jnp.float32),
            pltpu.VMEM((b, q), jnp.float32),
            pltpu.VMEM((N_PEERS, b, q), jnp.float32),
            pltpu.SemaphoreType.DMA((N_PEERS,)),
            pltpu.SemaphoreType.DMA((N_PEERS,)),
            pltpu.SemaphoreType.DMA((N_PEERS,)),
            pltpu.SemaphoreType.DMA((N_PEERS,)),
        ],
        compiler_params=pltpu.CompilerParams(
            collective_id=0,
            vmem_limit_bytes=100 * 1024 * 1024,
        ),
    )(x, Win0, Wout0, Win1, Wout1, Win2, Wout2)


# device time: 49954 ns/iter; 1.8922x vs baseline; 1.1134x over previous
import jax
import jax.numpy as jnp
from jax import lax
from jax.experimental import pallas as pl
from jax.experimental.pallas import tpu as pltpu

N_DEV = 4
N_PEERS = N_DEV - 1
N_LAYERS = 3
N_HALF = 2
SEND_ORDER = (2, 1, 3)


def kernel(x, Win0, Wout0, Win1, Wout1, Win2, Wout2):
    b, d_shard = x.shape
    h_dim = Win0.shape[1]
    out_cols = Wout0.shape[1]
    q = h_dim // N_DEV
    hw = q // N_HALF

    def body(x_ref, win0_ref, wout0_ref, win1_ref, wout1_ref,
             win2_ref, wout2_ref, out_ref,
             partial, rs_buf, myq, ag_buf,
             rs_send_sems, rs_recv_sems, ag_send_sems, ag_recv_sems):
        my = lax.axis_index("i")
        peers = [lax.rem(my + o, N_DEV) for o in range(1, N_DEV)]

        barrier = pltpu.get_barrier_semaphore()
        for p in peers:
            pl.semaphore_signal(barrier, inc=1, device_id=(p,),
                                device_id_type=pl.DeviceIdType.MESH)
        pl.semaphore_wait(barrier, N_PEERS)

        wins = [win0_ref, win1_ref, win2_ref]
        wouts = [wout0_ref, wout1_ref, wout2_ref]

        def rs_recv_desc(k, half):
            return pltpu.make_async_remote_copy(
                src_ref=rs_buf.at[k, :, pl.ds(half * hw, hw)],
                dst_ref=rs_buf.at[k, :, pl.ds(half * hw, hw)],
                send_sem=rs_send_sems.at[k, half],
                recv_sem=rs_recv_sems.at[k, half],
                device_id=(my,), device_id_type=pl.DeviceIdType.MESH)

        def ag_recv_desc(k, half):
            return pltpu.make_async_remote_copy(
                src_ref=ag_buf.at[k, :, pl.ds(half * hw, hw)],
                dst_ref=ag_buf.at[k, :, pl.ds(half * hw, hw)],
                send_sem=ag_send_sems.at[k, half],
                recv_sem=ag_recv_sems.at[k, half],
                device_id=(my,), device_id_type=pl.DeviceIdType.MESH)

        x_val = x_ref[...]
        for l in range(N_LAYERS):
            win, wout = wins[l], wouts[l]

            rs_sends = []
            for o in SEND_ORDER:
                p = peers[o - 1]
                partial[p] = jnp.dot(x_val, win[:, pl.ds(p * q, q)],
                                     preferred_element_type=jnp.float32)
                for half in range(N_HALF):
                    rdma = pltpu.make_async_remote_copy(
                        src_ref=partial.at[p, :, pl.ds(half * hw, hw)],
                        dst_ref=rs_buf.at[N_PEERS - o, :, pl.ds(half * hw, hw)],
                        send_sem=rs_send_sems.at[o - 1, half],
                        recv_sem=rs_recv_sems.at[N_PEERS - o, half],
                        device_id=(p,), device_id_type=pl.DeviceIdType.MESH)
                    rdma.start()
                    rs_sends.append(rdma)
            own_q = jnp.dot(x_val, win[:, pl.ds(my * q, q)],
                            preferred_element_type=jnp.float32)

            x_val = jnp.zeros((b, out_cols), jnp.float32)
            ag_sends = []
            for half in range(N_HALF):
                sl = slice(half * hw, (half + 1) * hw)
                acc = own_q[:, sl]
                for k in range(N_PEERS):
                    rs_recv_desc(k, half).wait_recv()
                    acc = acc + rs_buf[k, :, sl]
                myq[:, sl] = jnp.maximum(acc, 0.0)
                for o in SEND_ORDER:
                    rdma = pltpu.make_async_remote_copy(
                        src_ref=myq.at[:, pl.ds(half * hw, hw)],
                        dst_ref=ag_buf.at[N_PEERS - o, :, pl.ds(half * hw, hw)],
                        send_sem=ag_send_sems.at[o - 1, half],
                        recv_sem=ag_recv_sems.at[N_PEERS - o, half],
                        device_id=(peers[o - 1],),
                        device_id_type=pl.DeviceIdType.MESH)
                    rdma.start()
                    ag_sends.append(rdma)
                x_val = x_val + jnp.dot(
                    myq[:, sl], wout[pl.ds(my * q + half * hw, hw), :],
                    preferred_element_type=jnp.float32)

            for half in range(N_HALF):
                for k in range(N_PEERS):
                    ag_recv_desc(k, half).wait_recv()
                    s = lax.rem(my + k + 1, N_DEV)
                    x_val = x_val + jnp.dot(
                        ag_buf[k, :, slice(half * hw, (half + 1) * hw)],
                        wout[pl.ds(s * q + half * hw, hw), :],
                        preferred_element_type=jnp.float32)

            for rdma in rs_sends + ag_sends:
                rdma.wait_send()

        out_ref[...] = x_val

    return pl.pallas_call(
        body,
        out_shape=jax.ShapeDtypeStruct((b, out_cols), jnp.float32),
        in_specs=[pl.BlockSpec(memory_space=pltpu.VMEM)] * 7,
        out_specs=pl.BlockSpec(memory_space=pltpu.VMEM),
        scratch_shapes=[
            pltpu.VMEM((N_DEV, b, q), jnp.float32),
            pltpu.VMEM((N_PEERS, b, q), jnp.float32),
            pltpu.VMEM((b, q), jnp.float32),
            pltpu.VMEM((N_PEERS, b, q), jnp.float32),
            pltpu.SemaphoreType.DMA((N_PEERS, N_HALF)),
            pltpu.SemaphoreType.DMA((N_PEERS, N_HALF)),
            pltpu.SemaphoreType.DMA((N_PEERS, N_HALF)),
            pltpu.SemaphoreType.DMA((N_PEERS, N_HALF)),
        ],
        compiler_params=pltpu.CompilerParams(
            collective_id=0,
            vmem_limit_bytes=100 * 1024 * 1024,
        ),
    )(x, Win0, Wout0, Win1, Wout1, Win2, Wout2)


# device time: 30541 ns/iter; 3.0950x vs baseline; 1.6356x over previous
import jax
import jax.numpy as jnp
from jax import lax
from jax.experimental import pallas as pl
from jax.experimental.pallas import tpu as pltpu

N_DEV = 4
N_PEERS = N_DEV - 1
N_LAYERS = 3
N_HALF = 4
SEND_ORDER = (2, 1, 3)


def kernel(x, Win0, Wout0, Win1, Wout1, Win2, Wout2):
    b, d_shard = x.shape
    h_dim = Win0.shape[1]
    out_cols = Wout0.shape[1]
    q = h_dim // N_DEV
    hw = q // N_HALF

    def body(x_ref, win0_ref, wout0_ref, win1_ref, wout1_ref,
             win2_ref, wout2_ref, out_ref,
             win_vmem, wout_vmem, partial, rs_buf, myq, ag_buf,
             w_sems, rs_send_sems, rs_recv_sems, ag_send_sems, ag_recv_sems):
        my = lax.axis_index("i")
        peers = [lax.rem(my + o, N_DEV) for o in range(1, N_DEV)]

        win_hbm = [win0_ref, win1_ref, win2_ref]
        wout_hbm = [wout0_ref, wout1_ref, wout2_ref]

        def win_copy(l):
            return pltpu.make_async_copy(
                win_hbm[l], win_vmem.at[l % 2], w_sems.at[l % 2, 0])

        def wout_copy(l):
            return pltpu.make_async_copy(
                wout_hbm[l], wout_vmem.at[l % 2], w_sems.at[l % 2, 1])

        for l in (0, 1):
            win_copy(l).start()
            wout_copy(l).start()

        barrier = pltpu.get_barrier_semaphore()
        for p in peers:
            pl.semaphore_signal(barrier, inc=1, device_id=(p,),
                                device_id_type=pl.DeviceIdType.MESH)
        pl.semaphore_wait(barrier, N_PEERS)

        def rs_recv_desc(k, half):
            return pltpu.make_async_remote_copy(
                src_ref=rs_buf.at[k, :, pl.ds(half * hw, hw)],
                dst_ref=rs_buf.at[k, :, pl.ds(half * hw, hw)],
                send_sem=rs_send_sems.at[k, half],
                recv_sem=rs_recv_sems.at[k, half],
                device_id=(my,), device_id_type=pl.DeviceIdType.MESH)

        def ag_recv_desc(k, half):
            return pltpu.make_async_remote_copy(
                src_ref=ag_buf.at[k, :, pl.ds(half * hw, hw)],
                dst_ref=ag_buf.at[k, :, pl.ds(half * hw, hw)],
                send_sem=ag_send_sems.at[k, half],
                recv_sem=ag_recv_sems.at[k, half],
                device_id=(my,), device_id_type=pl.DeviceIdType.MESH)

        x_val = x_ref[...]
        for l in range(N_LAYERS):
            slot = l % 2
            if l == 1:
                win_copy(2).start()
                wout_copy(2).start()
            win_copy(l).wait()
            win = win_vmem.at[slot]
            wout = wout_vmem.at[slot]

            rs_sends = []
            for o in SEND_ORDER:
                p = peers[o - 1]
                partial[p] = jnp.dot(
                    x_val, win[:, pl.ds(p * q, q)],
                    preferred_element_type=jnp.float32).astype(jnp.bfloat16)
                for half in range(N_HALF):
                    rdma = pltpu.make_async_remote_copy(
                        src_ref=partial.at[p, :, pl.ds(half * hw, hw)],
                        dst_ref=rs_buf.at[N_PEERS - o, :, pl.ds(half * hw, hw)],
                        send_sem=rs_send_sems.at[o - 1, half],
                        recv_sem=rs_recv_sems.at[N_PEERS - o, half],
                        device_id=(p,), device_id_type=pl.DeviceIdType.MESH)
                    rdma.start()
                    rs_sends.append(rdma)
            own_q = jnp.dot(x_val, win[:, pl.ds(my * q, q)],
                            preferred_element_type=jnp.float32)
            wout_copy(l).wait()

            x_val = jnp.zeros((b, out_cols), jnp.float32)
            ag_sends = []
            for half in range(N_HALF):
                sl = slice(half * hw, (half + 1) * hw)
                acc = own_q[:, sl]
                for k in (1, 2, 0):
                    rs_recv_desc(k, half).wait_recv()
                    acc = acc + rs_buf[k, :, sl].astype(jnp.float32)
                relu_h = jnp.maximum(acc, 0.0)
                myq[:, sl] = relu_h.astype(jnp.bfloat16)
                for o in SEND_ORDER:
                    rdma = pltpu.make_async_remote_copy(
                        src_ref=myq.at[:, pl.ds(half * hw, hw)],
                        dst_ref=ag_buf.at[N_PEERS - o, :, pl.ds(half * hw, hw)],
                        send_sem=ag_send_sems.at[o - 1, half],
                        recv_sem=ag_recv_sems.at[N_PEERS - o, half],
                        device_id=(peers[o - 1],),
                        device_id_type=pl.DeviceIdType.MESH)
                    rdma.start()
                    ag_sends.append(rdma)
                x_val = x_val + jnp.dot(
                    relu_h, wout[pl.ds(my * q + half * hw, hw), :],
                    preferred_element_type=jnp.float32)

            for half in range(N_HALF):
                for k in (1, 2, 0):
                    ag_recv_desc(k, half).wait_recv()
                    s = lax.rem(my + k + 1, N_DEV)
                    x_val = x_val + jnp.dot(
                        ag_buf[k, :, slice(half * hw, (half + 1) * hw)]
                        .astype(jnp.float32),
                        wout[pl.ds(s * q + half * hw, hw), :],
                        preferred_element_type=jnp.float32)

            for rdma in rs_sends + ag_sends:
                rdma.wait_send()

        out_ref[...] = x_val

    weight_spec = pl.BlockSpec(memory_space=pl.ANY)
    return pl.pallas_call(
        body,
        out_shape=jax.ShapeDtypeStruct((b, out_cols), jnp.float32),
        in_specs=[pl.BlockSpec(memory_space=pltpu.VMEM)] + [weight_spec] * 6,
        out_specs=pl.BlockSpec(memory_space=pltpu.VMEM),
        scratch_shapes=[
            pltpu.VMEM((2, d_shard, h_dim), jnp.float32),
            pltpu.VMEM((2, h_dim, out_cols), jnp.float32),
            pltpu.VMEM((N_DEV, b, q), jnp.bfloat16),
            pltpu.VMEM((N_PEERS, b, q), jnp.bfloat16),
            pltpu.VMEM((b, q), jnp.bfloat16),
            pltpu.VMEM((N_PEERS, b, q), jnp.bfloat16),
            pltpu.SemaphoreType.DMA((2, 2)),
            pltpu.SemaphoreType.DMA((N_PEERS, N_HALF)),
            pltpu.SemaphoreType.DMA((N_PEERS, N_HALF)),
            pltpu.SemaphoreType.DMA((N_PEERS, N_HALF)),
            pltpu.SemaphoreType.DMA((N_PEERS, N_HALF)),
        ],
        compiler_params=pltpu.CompilerParams(
            collective_id=0,
            vmem_limit_bytes=100 * 1024 * 1024,
        ),
    )(x, Win0, Wout0, Win1, Wout1, Win2, Wout2)
